# baseline (speedup 1.0000x reference)
import functools

import jax
import jax.numpy as jnp
from jax import lax
from jax.experimental import pallas as pl
from jax.experimental.pallas import tpu as pltpu
from jax.experimental.pallas import tpu_sc as plsc

B, L = 4096, 200
D = 128
OUT = 512
PVOC = 1000
NC, NS = 2, 16
NW = NC * NS
RPW = B // NW
C0, C1 = 96, 104
LANES = 16
DV = D // LANES
UNROLL = 4
G = 8
NG = RPW // G


def _sc_embed(x_p, x_v, P_table, V_table):
    mesh = plsc.VectorSubcoreMesh(
        core_axis_name="c", subcore_axis_name="s", num_cores=NC, num_subcores=NS
    )

    @functools.partial(
        pl.kernel,
        mesh=mesh,
        out_type=jax.ShapeDtypeStruct((B, 2 * D), jnp.float32),
        scratch_types=[
            pltpu.VMEM((RPW * L,), jnp.int32),
            pltpu.VMEM((RPW * L,), jnp.int32),
            pltpu.VMEM((C0, D), jnp.float32),
            pltpu.VMEM((C1, D), jnp.float32),
            pltpu.VMEM((C0, D), jnp.float32),
            pltpu.VMEM((C1, D), jnp.float32),
            pltpu.VMEM((2, G, 2 * D), jnp.float32),
            pltpu.VMEM((2, 8, L), jnp.int32),
            pltpu.VMEM_SHARED((PVOC, D), jnp.float32),
            pltpu.SemaphoreType.DMA,
            pltpu.SemaphoreType.DMA,
            pltpu.SemaphoreType.DMA,
            pltpu.SemaphoreType.DMA,
            pltpu.SemaphoreType.DMA,
            pltpu.SemaphoreType.DMA,
        ],
    )
    def body(xp_hbm, xv_hbm, p_hbm, v_hbm, out_hbm,
             idxp, idxv, bpa, bpb, bva, bvb, ostage, tstage, psp,
             pas, pbs, vas, vbs, osem, tsem):
        c = lax.axis_index("c")
        s = lax.axis_index("s")
        wid = c * NS + s
        base = wid * RPW

        COFFS = tuple(range(0, 192, 16)) + (L - 16,)

        def stage_flat(x2d, flat):
            def fire_t(k, slot):
                src = x2d.at[pl.ds(pl.multiple_of(base + k * 8, 8), 8)]
                pltpu.async_copy(src, tstage.at[slot], tsem)

            fire_t(0, 0)

            def blk(k, carry):
                slot = k % 2
                pltpu.make_async_copy(x2d.at[pl.ds(0, 8)], tstage.at[0],
                                      tsem).wait()
                fire_t(jnp.minimum(k + 1, RPW // 8 - 1), (k + 1) % 2)
                for r in range(8):
                    fb = (k * 8 + r) * L
                    for off in COFFS:
                        flat[pl.ds(pl.multiple_of(fb + off, 8), 16)] = \
                            tstage[slot, r, pl.ds(off, 16)]
                return carry

            lax.fori_loop(0, RPW // 8, blk, 0)
            pltpu.make_async_copy(x2d.at[pl.ds(0, 8)], tstage.at[0],
                                  tsem).wait()

        stage_flat(xp_hbm, idxp)
        stage_flat(xv_hbm, idxv)

        po = pl.multiple_of(s * 64, 8)

        @pl.when(s < NS - 1)
        def _():
            pltpu.sync_copy(p_hbm.at[pl.ds(po, 64)], psp.at[pl.ds(po, 64)])

        @pl.when(s == NS - 1)
        def _():
            pltpu.sync_copy(p_hbm.at[pl.ds(960, 40)], psp.at[pl.ds(960, 40)])
        plsc.subcore_barrier()

        def fire(tab, idx, i, buf, sem, lo, n):
            o = pl.multiple_of(i * L + lo, 8)
            pltpu.async_copy(tab.at[idx.at[pl.ds(o, n)]], buf, sem)

        def drain(tab, buf, sem):
            pltpu.make_async_copy(tab.at[pl.ds(0, buf.shape[0])], buf,
                                  sem).wait()

        def drain_out():
            pltpu.make_async_copy(ostage.at[0],
                                  out_hbm.at[pl.ds(0, G)], osem).wait()

        def accum(buf, n, init):
            def red(j4, accs):
                j = j4 * UNROLL
                for u in range(UNROLL):
                    accs = tuple(a + buf[j + u, pl.ds(cc * LANES, LANES)]
                                 for cc, a in enumerate(accs))
                return accs
            return lax.fori_loop(0, n // UNROLL, red, init)

        zeros8 = tuple(jnp.zeros((LANES,), jnp.float32) for _ in range(DV))

        def fire_all(i):
            fire(psp, idxp, i, bpa, pas, 0, C0)
            fire(psp, idxp, i, bpb, pbs, C0, C1)
            fire(v_hbm, idxv, i, bva, vas, 0, C0)
            fire(v_hbm, idxv, i, bvb, vbs, C0, C1)

        fire_all(0)

        def group(g, carry):
            gp = g % 2
            for k in range(G):
                i = g * G + k
                nxt = jnp.minimum(i + 1, RPW - 1)
                drain(psp, bpa, pas)
                pacc = accum(bpa, C0, zeros8)
                fire(psp, idxp, nxt, bpa, pas, 0, C0)
                drain(psp, bpb, pbs)
                pacc = accum(bpb, C1, pacc)
                fire(psp, idxp, nxt, bpb, pbs, C0, C1)
                drain(v_hbm, bva, vas)
                vacc = accum(bva, C0, zeros8)
                fire(v_hbm, idxv, nxt, bva, vas, 0, C0)
                drain(v_hbm, bvb, vbs)
                vacc = accum(bvb, C1, vacc)
                fire(v_hbm, idxv, nxt, bvb, vbs, C0, C1)
                for cc in range(DV):
                    ostage[gp, k, pl.ds(cc * LANES, LANES)] = pacc[cc]
                    ostage[gp, k, pl.ds(D + cc * LANES, LANES)] = vacc[cc]
            oof = pl.multiple_of(base + g * G, 8)
            pltpu.async_copy(ostage.at[gp], out_hbm.at[pl.ds(oof, G)], osem)

            @pl.when(g >= 2)
            def _():
                drain_out()
            return carry

        lax.fori_loop(0, NG, group, 0)
        drain(psp, bpa, pas)
        drain(psp, bpb, pbs)
        drain(v_hbm, bva, vas)
        drain(v_hbm, bvb, vbs)
        drain_out()
        drain_out()

    return body(x_p, x_v, P_table, V_table)


def _tc_head(sums, x_len, W, b2d):
    BT = 256

    def body(h_ref, len_ref, w_ref, b_ref, o_ref):
        h = h_ref[...] / len_ref[...].astype(jnp.float32)
        res = jnp.dot(h, w_ref[...], preferred_element_type=jnp.float32)
        res = res + b_ref[...]
        m = jnp.max(res, axis=-1, keepdims=True)
        e = res - m
        lse = jnp.log(jnp.sum(jnp.exp(e), axis=-1, keepdims=True))
        o_ref[...] = e - lse

    return pl.pallas_call(
        body,
        grid=(B // BT,),
        in_specs=[
            pl.BlockSpec((BT, 2 * D), lambda i: (i, 0)),
            pl.BlockSpec((BT, 1), lambda i: (i, 0)),
            pl.BlockSpec((2 * D, OUT), lambda i: (0, 0)),
            pl.BlockSpec((1, OUT), lambda i: (0, 0)),
        ],
        out_specs=pl.BlockSpec((BT, OUT), lambda i: (i, 0)),
        out_shape=jax.ShapeDtypeStruct((B, OUT), jnp.float32),
    )(sums, x_len, W, b2d)


@jax.jit
def kernel(x_p, x_v, x_len, P_table, V_table, W, b):
    sums = _sc_embed(x_p, x_v, P_table, V_table)
    return _tc_head(sums, x_len, W, b.reshape(1, OUT))


# speedup vs baseline: 1.0266x; 1.0266x over previous
import functools

import jax
import jax.numpy as jnp
from jax import lax
from jax.experimental import pallas as pl
from jax.experimental.pallas import tpu as pltpu
from jax.experimental.pallas import tpu_sc as plsc

B, L = 4096, 200
D = 128
OUT = 512
PVOC = 1000
NC, NS = 2, 16
NW = NC * NS
RPW = B // NW
C0, C1 = 96, 104
LANES = 16
DV = D // LANES
UNROLL = 4
G = 8
NG = RPW // G


def _sc_embed(x_p, x_v, P_table, V_table):
    mesh = plsc.VectorSubcoreMesh(
        core_axis_name="c", subcore_axis_name="s", num_cores=NC, num_subcores=NS
    )

    @functools.partial(
        pl.kernel,
        mesh=mesh,
        out_type=jax.ShapeDtypeStruct((B, 2 * D), jnp.float32),
        scratch_types=[
            pltpu.VMEM((RPW * L,), jnp.int32),
            pltpu.VMEM((RPW * L,), jnp.int32),
            pltpu.VMEM((C0, D), jnp.float32),
            pltpu.VMEM((C1, D), jnp.float32),
            pltpu.VMEM((C0, D), jnp.float32),
            pltpu.VMEM((C1, D), jnp.float32),
            pltpu.VMEM((2, G, 2 * D), jnp.float32),
            pltpu.VMEM((8, 8, L), jnp.int32),
            pltpu.VMEM_SHARED((PVOC, D), jnp.float32),
            pltpu.SemaphoreType.DMA,
            pltpu.SemaphoreType.DMA,
            pltpu.SemaphoreType.DMA,
            pltpu.SemaphoreType.DMA,
            pltpu.SemaphoreType.DMA,
            pltpu.SemaphoreType.DMA,
        ],
    )
    def body(xp_hbm, xv_hbm, p_hbm, v_hbm, out_hbm,
             idxp, idxv, bpa, bpb, bva, bvb, ostage, tstage, psp,
             pas, pbs, vas, vbs, osem, tsem):
        c = lax.axis_index("c")
        s = lax.axis_index("s")
        wid = c * NS + s
        base = wid * RPW

        COFFS = tuple(range(0, 192, 16)) + (L - 16,)

        TDEPTH = 7

        def stage_flat(x2d, flat):
            def fire_t(k, slot):
                src = x2d.at[pl.ds(pl.multiple_of(base + k * 8, 8), 8)]
                pltpu.async_copy(src, tstage.at[slot], tsem)

            for k in range(TDEPTH):
                fire_t(k, k)

            def blk(k, carry):
                slot = k % 8
                pltpu.make_async_copy(x2d.at[pl.ds(0, 8)], tstage.at[0],
                                      tsem).wait()
                fire_t(jnp.minimum(k + TDEPTH, RPW // 8 - 1),
                       (k + TDEPTH) % 8)
                for r in range(8):
                    fb = (k * 8 + r) * L
                    for off in COFFS:
                        flat[pl.ds(pl.multiple_of(fb + off, 8), 16)] = \
                            tstage[slot, r, pl.ds(off, 16)]
                return carry

            lax.fori_loop(0, RPW // 8, blk, 0)
            for _ in range(TDEPTH):
                pltpu.make_async_copy(x2d.at[pl.ds(0, 8)], tstage.at[0],
                                      tsem).wait()

        stage_flat(xp_hbm, idxp)
        stage_flat(xv_hbm, idxv)

        po = pl.multiple_of(s * 64, 8)

        @pl.when(s < NS - 1)
        def _():
            pltpu.sync_copy(p_hbm.at[pl.ds(po, 64)], psp.at[pl.ds(po, 64)])

        @pl.when(s == NS - 1)
        def _():
            pltpu.sync_copy(p_hbm.at[pl.ds(960, 40)], psp.at[pl.ds(960, 40)])
        plsc.subcore_barrier()

        def fire(tab, idx, i, buf, sem, lo, n):
            o = pl.multiple_of(i * L + lo, 8)
            pltpu.async_copy(tab.at[idx.at[pl.ds(o, n)]], buf, sem)

        def drain(tab, buf, sem):
            pltpu.make_async_copy(tab.at[pl.ds(0, buf.shape[0])], buf,
                                  sem).wait()

        def drain_out():
            pltpu.make_async_copy(ostage.at[0],
                                  out_hbm.at[pl.ds(0, G)], osem).wait()

        def accum(buf, n, init):
            def red(j4, accs):
                j = j4 * UNROLL
                for u in range(UNROLL):
                    accs = tuple(a + buf[j + u, pl.ds(cc * LANES, LANES)]
                                 for cc, a in enumerate(accs))
                return accs
            return lax.fori_loop(0, n // UNROLL, red, init)

        zeros8 = tuple(jnp.zeros((LANES,), jnp.float32) for _ in range(DV))

        def fire_all(i):
            fire(psp, idxp, i, bpa, pas, 0, C0)
            fire(psp, idxp, i, bpb, pbs, C0, C1)
            fire(v_hbm, idxv, i, bva, vas, 0, C0)
            fire(v_hbm, idxv, i, bvb, vbs, C0, C1)

        fire_all(0)

        def group(g, carry):
            gp = g % 2
            for k in range(G):
                i = g * G + k
                nxt = jnp.minimum(i + 1, RPW - 1)
                drain(psp, bpa, pas)
                pacc = accum(bpa, C0, zeros8)
                fire(psp, idxp, nxt, bpa, pas, 0, C0)
                drain(psp, bpb, pbs)
                pacc = accum(bpb, C1, pacc)
                fire(psp, idxp, nxt, bpb, pbs, C0, C1)
                drain(v_hbm, bva, vas)
                vacc = accum(bva, C0, zeros8)
                fire(v_hbm, idxv, nxt, bva, vas, 0, C0)
                drain(v_hbm, bvb, vbs)
                vacc = accum(bvb, C1, vacc)
                fire(v_hbm, idxv, nxt, bvb, vbs, C0, C1)
                for cc in range(DV):
                    ostage[gp, k, pl.ds(cc * LANES, LANES)] = pacc[cc]
                    ostage[gp, k, pl.ds(D + cc * LANES, LANES)] = vacc[cc]
            oof = pl.multiple_of(base + g * G, 8)
            pltpu.async_copy(ostage.at[gp], out_hbm.at[pl.ds(oof, G)], osem)

            @pl.when(g >= 2)
            def _():
                drain_out()
            return carry

        lax.fori_loop(0, NG, group, 0)
        drain(psp, bpa, pas)
        drain(psp, bpb, pbs)
        drain(v_hbm, bva, vas)
        drain(v_hbm, bvb, vbs)
        drain_out()
        drain_out()

    return body(x_p, x_v, P_table, V_table)


def _tc_head(sums, x_len, W, b2d):
    BT = 256

    def body(h_ref, len_ref, w_ref, b_ref, o_ref):
        h = h_ref[...] / len_ref[...].astype(jnp.float32)
        res = jnp.dot(h, w_ref[...], preferred_element_type=jnp.float32)
        res = res + b_ref[...]
        m = jnp.max(res, axis=-1, keepdims=True)
        e = res - m
        lse = jnp.log(jnp.sum(jnp.exp(e), axis=-1, keepdims=True))
        o_ref[...] = e - lse

    return pl.pallas_call(
        body,
        grid=(B // BT,),
        in_specs=[
            pl.BlockSpec((BT, 2 * D), lambda i: (i, 0)),
            pl.BlockSpec((BT, 1), lambda i: (i, 0)),
            pl.BlockSpec((2 * D, OUT), lambda i: (0, 0)),
            pl.BlockSpec((1, OUT), lambda i: (0, 0)),
        ],
        out_specs=pl.BlockSpec((BT, OUT), lambda i: (i, 0)),
        out_shape=jax.ShapeDtypeStruct((B, OUT), jnp.float32),
    )(sums, x_len, W, b2d)


@jax.jit
def kernel(x_p, x_v, x_len, P_table, V_table, W, b):
    sums = _sc_embed(x_p, x_v, P_table, V_table)
    return _tc_head(sums, x_len, W, b.reshape(1, OUT))


# speedup vs baseline: 1.0432x; 1.0162x over previous
import functools

import jax
import jax.numpy as jnp
from jax import lax
from jax.experimental import pallas as pl
from jax.experimental.pallas import tpu as pltpu
from jax.experimental.pallas import tpu_sc as plsc

B, L = 4096, 200
D = 128
OUT = 512
PVOC = 1000
NC, NS = 2, 16
NW = NC * NS
RPW = B // NW
C0, C1 = 96, 104
LANES = 16
DV = D // LANES
UNROLL = 4
G = 8
NG = RPW // G


def _sc_embed(x_p, x_v, P_table, V_table):
    mesh = plsc.VectorSubcoreMesh(
        core_axis_name="c", subcore_axis_name="s", num_cores=NC, num_subcores=NS
    )

    @functools.partial(
        pl.kernel,
        mesh=mesh,
        out_type=jax.ShapeDtypeStruct((B, 2 * D), jnp.float32),
        scratch_types=[
            pltpu.VMEM((RPW * L,), jnp.int32),
            pltpu.VMEM((RPW * L,), jnp.int32),
            pltpu.VMEM((C0, D), jnp.float32),
            pltpu.VMEM((C1, D), jnp.float32),
            pltpu.VMEM((C0, D), jnp.float32),
            pltpu.VMEM((C1, D), jnp.float32),
            pltpu.VMEM((2, G, 2 * D), jnp.float32),
            pltpu.VMEM((8, 8, L), jnp.int32),
            pltpu.VMEM_SHARED((PVOC, D), jnp.float32),
            pltpu.SemaphoreType.DMA,
            pltpu.SemaphoreType.DMA,
            pltpu.SemaphoreType.DMA,
            pltpu.SemaphoreType.DMA,
            pltpu.SemaphoreType.DMA,
            pltpu.SemaphoreType.DMA,
        ],
    )
    def body(xp_hbm, xv_hbm, p_hbm, v_hbm, out_hbm,
             idxp, idxv, bpa, bpb, bva, bvb, ostage, tstage, psp,
             pas, pbs, vas, vbs, osem, tsem):
        c = lax.axis_index("c")
        s = lax.axis_index("s")
        wid = c * NS + s
        base = wid * RPW

        COFFS = tuple(range(0, 192, 16)) + (L - 16,)

        TDEPTH = 7

        def stage_flat(x2d, flat):
            def fire_t(k, slot):
                src = x2d.at[pl.ds(pl.multiple_of(base + k * 8, 8), 8)]
                pltpu.async_copy(src, tstage.at[slot], tsem)

            for k in range(TDEPTH):
                fire_t(k, k)

            def blk(k, carry):
                slot = k % 8
                pltpu.make_async_copy(x2d.at[pl.ds(0, 8)], tstage.at[0],
                                      tsem).wait()
                fire_t(jnp.minimum(k + TDEPTH, RPW // 8 - 1),
                       (k + TDEPTH) % 8)
                for r in range(8):
                    fb = (k * 8 + r) * L
                    for off in COFFS:
                        flat[pl.ds(pl.multiple_of(fb + off, 8), 16)] = \
                            tstage[slot, r, pl.ds(off, 16)]
                return carry

            lax.fori_loop(0, RPW // 8, blk, 0)
            for _ in range(TDEPTH):
                pltpu.make_async_copy(x2d.at[pl.ds(0, 8)], tstage.at[0],
                                      tsem).wait()

        stage_flat(xp_hbm, idxp)
        stage_flat(xv_hbm, idxv)

        po = pl.multiple_of(s * 64, 8)

        @pl.when(s < NS - 1)
        def _():
            pltpu.sync_copy(p_hbm.at[pl.ds(po, 64)], psp.at[pl.ds(po, 64)])

        @pl.when(s == NS - 1)
        def _():
            pltpu.sync_copy(p_hbm.at[pl.ds(960, 40)], psp.at[pl.ds(960, 40)])
        plsc.subcore_barrier()

        def fire(tab, idx, i, buf, sem, lo, n):
            o = pl.multiple_of(i * L + lo, 8)
            pltpu.async_copy(tab.at[idx.at[pl.ds(o, n)]], buf, sem)

        def drain(tab, buf, sem):
            pltpu.make_async_copy(tab.at[pl.ds(0, buf.shape[0])], buf,
                                  sem).wait()

        def drain_out():
            pltpu.make_async_copy(ostage.at[0],
                                  out_hbm.at[pl.ds(0, G)], osem).wait()

        def accum(buf, n, init):
            def red(j4, accs):
                j = j4 * UNROLL
                for u in range(UNROLL):
                    accs = tuple(a + buf[j + u, pl.ds(cc * LANES, LANES)]
                                 for cc, a in enumerate(accs))
                return accs
            return lax.fori_loop(0, n // UNROLL, red, init)

        zeros8 = tuple(jnp.zeros((LANES,), jnp.float32) for _ in range(DV))

        def fire_all(i):
            fire(psp, idxp, i, bpa, pas, 0, C0)
            fire(psp, idxp, i, bpb, pbs, C0, C1)
            fire(v_hbm, idxv, i, bva, vas, 0, C0)
            fire(v_hbm, idxv, i, bvb, vbs, C0, C1)

        fire_all(0)

        def group(g, carry):
            gp = g % 2
            for k in range(G):
                i = g * G + k
                nxt = jnp.minimum(i + 1, RPW - 1)
                drain(psp, bpa, pas)
                pacc = accum(bpa, C0, zeros8)
                fire(psp, idxp, nxt, bpa, pas, 0, C0)
                drain(psp, bpb, pbs)
                pacc = accum(bpb, C1, pacc)
                fire(psp, idxp, nxt, bpb, pbs, C0, C1)
                drain(v_hbm, bva, vas)
                vacc = accum(bva, C0, zeros8)
                fire(v_hbm, idxv, nxt, bva, vas, 0, C0)
                drain(v_hbm, bvb, vbs)
                vacc = accum(bvb, C1, vacc)
                fire(v_hbm, idxv, nxt, bvb, vbs, C0, C1)
                for cc in range(DV):
                    ostage[gp, k, pl.ds(cc * LANES, LANES)] = pacc[cc]
                    ostage[gp, k, pl.ds(D + cc * LANES, LANES)] = vacc[cc]
            oof = pl.multiple_of(base + g * G, 8)
            pltpu.async_copy(ostage.at[gp], out_hbm.at[pl.ds(oof, G)], osem)

            @pl.when(g >= 2)
            def _():
                drain_out()
            return carry

        lax.fori_loop(0, NG, group, 0)
        drain(psp, bpa, pas)
        drain(psp, bpb, pbs)
        drain(v_hbm, bva, vas)
        drain(v_hbm, bvb, vbs)
        drain_out()
        drain_out()

    return body(x_p, x_v, P_table, V_table)


def _tc_head(sums, x_len, W, b2d):
    BT = 512

    def body(h_ref, len_ref, w_ref, b_ref, o_ref):
        h = h_ref[...] / len_ref[...].astype(jnp.float32)
        res = jnp.dot(h, w_ref[...], preferred_element_type=jnp.float32)
        res = res + b_ref[...]
        m = jnp.max(res, axis=-1, keepdims=True)
        e = res - m
        lse = jnp.log(jnp.sum(jnp.exp(e), axis=-1, keepdims=True))
        o_ref[...] = e - lse

    return pl.pallas_call(
        body,
        grid=(B // BT,),
        in_specs=[
            pl.BlockSpec((BT, 2 * D), lambda i: (i, 0)),
            pl.BlockSpec((BT, 1), lambda i: (i, 0)),
            pl.BlockSpec((2 * D, OUT), lambda i: (0, 0)),
            pl.BlockSpec((1, OUT), lambda i: (0, 0)),
        ],
        out_specs=pl.BlockSpec((BT, OUT), lambda i: (i, 0)),
        out_shape=jax.ShapeDtypeStruct((B, OUT), jnp.float32),
    )(sums, x_len, W, b2d)


@jax.jit
def kernel(x_p, x_v, x_len, P_table, V_table, W, b):
    sums = _sc_embed(x_p, x_v, P_table, V_table)
    return _tc_head(sums, x_len, W, b.reshape(1, OUT))


# speedup vs baseline: 1.0505x; 1.0070x over previous
import functools

import jax
import jax.numpy as jnp
from jax import lax
from jax.experimental import pallas as pl
from jax.experimental.pallas import tpu as pltpu
from jax.experimental.pallas import tpu_sc as plsc

B, L = 4096, 200
D = 128
OUT = 512
PVOC = 1000
NC, NS = 2, 16
NW = NC * NS
RPW = B // NW
C0, C1 = 96, 104
LANES = 16
DV = D // LANES
UNROLL = 4
G = 8
NG = RPW // G


def _sc_embed(x_p, x_v, P_table, V_table):
    mesh = plsc.VectorSubcoreMesh(
        core_axis_name="c", subcore_axis_name="s", num_cores=NC, num_subcores=NS
    )

    @functools.partial(
        pl.kernel,
        mesh=mesh,
        out_type=jax.ShapeDtypeStruct((B, 2 * D), jnp.float32),
        scratch_types=[
            pltpu.VMEM((RPW * L,), jnp.int32),
            pltpu.VMEM((RPW * L,), jnp.int32),
            pltpu.VMEM((C0, D), jnp.float32),
            pltpu.VMEM((C1, D), jnp.float32),
            pltpu.VMEM((C0, D), jnp.float32),
            pltpu.VMEM((C1, D), jnp.float32),
            pltpu.VMEM((2, G, 2 * D), jnp.float32),
            pltpu.VMEM((8, 8, L), jnp.int32),
            pltpu.VMEM_SHARED((PVOC, D), jnp.float32),
            pltpu.SemaphoreType.DMA,
            pltpu.SemaphoreType.DMA,
            pltpu.SemaphoreType.DMA,
            pltpu.SemaphoreType.DMA,
            pltpu.SemaphoreType.DMA,
            pltpu.SemaphoreType.DMA,
        ],
    )
    def body(xp_hbm, xv_hbm, p_hbm, v_hbm, out_hbm,
             idxp, idxv, bpa, bpb, bva, bvb, ostage, tstage, psp,
             pas, pbs, vas, vbs, osem, tsem):
        c = lax.axis_index("c")
        s = lax.axis_index("s")
        wid = c * NS + s
        base = wid * RPW

        COFFS = tuple(range(0, 192, 16)) + (L - 16,)

        TDEPTH = 7

        def stage_flat(x2d, flat):
            def fire_t(k, slot):
                src = x2d.at[pl.ds(pl.multiple_of(base + k * 8, 8), 8)]
                pltpu.async_copy(src, tstage.at[slot], tsem)

            for k in range(TDEPTH):
                fire_t(k, k)

            def blk(k, carry):
                slot = k % 8
                pltpu.make_async_copy(x2d.at[pl.ds(0, 8)], tstage.at[0],
                                      tsem).wait()
                fire_t(jnp.minimum(k + TDEPTH, RPW // 8 - 1),
                       (k + TDEPTH) % 8)
                for r in range(8):
                    fb = (k * 8 + r) * L
                    for off in COFFS:
                        flat[pl.ds(pl.multiple_of(fb + off, 8), 16)] = \
                            tstage[slot, r, pl.ds(off, 16)]
                return carry

            lax.fori_loop(0, RPW // 8, blk, 0)
            for _ in range(TDEPTH):
                pltpu.make_async_copy(x2d.at[pl.ds(0, 8)], tstage.at[0],
                                      tsem).wait()

        stage_flat(xp_hbm, idxp)
        stage_flat(xv_hbm, idxv)

        po = pl.multiple_of(s * 64, 8)

        @pl.when(s < NS - 1)
        def _():
            pltpu.sync_copy(p_hbm.at[pl.ds(po, 64)], psp.at[pl.ds(po, 64)])

        @pl.when(s == NS - 1)
        def _():
            pltpu.sync_copy(p_hbm.at[pl.ds(960, 40)], psp.at[pl.ds(960, 40)])
        plsc.subcore_barrier()

        def fire(tab, idx, i, buf, sem, lo, n):
            o = pl.multiple_of(i * L + lo, 8)
            pltpu.async_copy(tab.at[idx.at[pl.ds(o, n)]], buf, sem)

        def drain(tab, buf, sem):
            pltpu.make_async_copy(tab.at[pl.ds(0, buf.shape[0])], buf,
                                  sem).wait()

        def drain_out():
            pltpu.make_async_copy(ostage.at[0],
                                  out_hbm.at[pl.ds(0, G)], osem).wait()

        def accum(buf, n, init):
            def red(j4, accs):
                j = j4 * UNROLL
                for u in range(UNROLL):
                    accs = tuple(a + buf[j + u, pl.ds(cc * LANES, LANES)]
                                 for cc, a in enumerate(accs))
                return accs
            return lax.fori_loop(0, n // UNROLL, red, init)

        zeros8 = tuple(jnp.zeros((LANES,), jnp.float32) for _ in range(DV))

        def fire_all(i):
            fire(psp, idxp, i, bpa, pas, 0, C0)
            fire(psp, idxp, i, bpb, pbs, C0, C1)
            fire(v_hbm, idxv, i, bva, vas, 0, C0)
            fire(v_hbm, idxv, i, bvb, vbs, C0, C1)

        fire_all(0)

        def group(g, carry):
            gp = g % 2
            for k in range(G):
                i = g * G + k
                nxt = jnp.minimum(i + 1, RPW - 1)
                drain(psp, bpa, pas)
                pacc = accum(bpa, C0, zeros8)
                fire(psp, idxp, nxt, bpa, pas, 0, C0)
                drain(psp, bpb, pbs)
                pacc = accum(bpb, C1, pacc)
                fire(psp, idxp, nxt, bpb, pbs, C0, C1)
                drain(v_hbm, bva, vas)
                vacc = accum(bva, C0, zeros8)
                fire(v_hbm, idxv, nxt, bva, vas, 0, C0)
                drain(v_hbm, bvb, vbs)
                vacc = accum(bvb, C1, vacc)
                fire(v_hbm, idxv, nxt, bvb, vbs, C0, C1)
                for cc in range(DV):
                    ostage[gp, k, pl.ds(cc * LANES, LANES)] = pacc[cc]
                    ostage[gp, k, pl.ds(D + cc * LANES, LANES)] = vacc[cc]
            oof = pl.multiple_of(base + g * G, 8)
            pltpu.async_copy(ostage.at[gp], out_hbm.at[pl.ds(oof, G)], osem)

            @pl.when(g >= 2)
            def _():
                drain_out()
            return carry

        lax.fori_loop(0, NG, group, 0)
        drain(psp, bpa, pas)
        drain(psp, bpb, pbs)
        drain(v_hbm, bva, vas)
        drain(v_hbm, bvb, vbs)
        drain_out()
        drain_out()

    return body(x_p, x_v, P_table, V_table)


def _tc_head(sums, x_len, W, b2d):
    BT = 1024

    def body(h_ref, len_ref, w_ref, b_ref, o_ref):
        h = h_ref[...] / len_ref[...].astype(jnp.float32)
        res = jnp.dot(h, w_ref[...], preferred_element_type=jnp.float32)
        res = res + b_ref[...]
        m = jnp.max(res, axis=-1, keepdims=True)
        e = res - m
        lse = jnp.log(jnp.sum(jnp.exp(e), axis=-1, keepdims=True))
        o_ref[...] = e - lse

    return pl.pallas_call(
        body,
        grid=(B // BT,),
        in_specs=[
            pl.BlockSpec((BT, 2 * D), lambda i: (i, 0)),
            pl.BlockSpec((BT, 1), lambda i: (i, 0)),
            pl.BlockSpec((2 * D, OUT), lambda i: (0, 0)),
            pl.BlockSpec((1, OUT), lambda i: (0, 0)),
        ],
        out_specs=pl.BlockSpec((BT, OUT), lambda i: (i, 0)),
        out_shape=jax.ShapeDtypeStruct((B, OUT), jnp.float32),
    )(sums, x_len, W, b2d)


@jax.jit
def kernel(x_p, x_v, x_len, P_table, V_table, W, b):
    sums = _sc_embed(x_p, x_v, P_table, V_table)
    return _tc_head(sums, x_len, W, b.reshape(1, OUT))
